# baseline (speedup 1.0000x reference)
import jax
import jax.numpy as jnp
from jax import lax
from jax.experimental import pallas as pl
from jax.experimental.pallas import tpu as pltpu
from jax.experimental.pallas import tpu_sc as plsc

_N = 100000
_D = 128
_S = 256
_B = 2000
_R = 128
_NCH = _N // _R
_TAILB = _NCH * _R
_TAIL = _N - _TAILB
_NW = 32


def _tc_scores(x_ref, w1_ref, b1_ref, w2_ref, e_ref):
    x = x_ref[...]
    h = jnp.tanh(
        jnp.dot(x, w1_ref[...], preferred_element_type=jnp.float32) + b1_ref[...]
    )
    st = lax.dot_general(
        w2_ref[...], h, (((1,), (1,)), ((), ())), preferred_element_type=jnp.float32
    )
    c = jnp.sum(jnp.abs(w2_ref[...]))
    e_ref[...] = jnp.exp(st - c).reshape(1, 1, _B)


def _sc_pool(x_hbm, e_hbm, b_hbm, acc_hbm, den_hbm,
             xv, ev, bv, dv, bt, zv, accs, dens):
    cid = lax.axis_index("c")
    sid = lax.axis_index("s")
    wid = sid * 2 + cid

    z16 = jnp.zeros((16,), jnp.float32)
    for r in range(16):
        for q in range(8):
            zv[r, pl.ds(16 * q, 16)] = z16
    pltpu.sync_copy(zv, accs.at[pl.ds(sid * 16, 16), :])
    pltpu.sync_copy(zv, dens.at[pl.ds(sid * 16, 16), :])
    plsc.subcore_barrier()

    nch_w = (_NCH - 1 - wid) // _NW + 1

    def weight_rows(nrows):
        def row_body(i, carry):
            ei = ev[pl.ds(i, 16)][0]
            es = jnp.full((16,), ei, jnp.float32)
            for q in range(8):
                sl = pl.ds(16 * q, 16)
                xv[i, sl] = xv[i, sl] * es
                dv[i, sl] = es
            return carry
        lax.fori_loop(0, nrows, row_body, 0)

    def chunk_body(k, carry):
        base = (wid + k * _NW) * _R
        pltpu.sync_copy(x_hbm.at[pl.ds(base, _R), :], xv)
        pltpu.sync_copy(e_hbm.at[pl.ds(base, _R)], ev.at[pl.ds(0, _R)])
        pltpu.sync_copy(b_hbm.at[pl.ds(base, _R)], bv)
        weight_rows(_R)
        pltpu.sync_copy(xv, accs.at[bv], add=True)
        pltpu.sync_copy(dv, dens.at[bv], add=True)
        return carry

    lax.fori_loop(0, nch_w, chunk_body, 0)

    @pl.when(wid == 0)
    def _tail():
        pltpu.sync_copy(x_hbm.at[pl.ds(_TAILB, _TAIL), :], xv.at[pl.ds(0, _TAIL), :])
        pltpu.sync_copy(e_hbm.at[pl.ds(_TAILB, _TAIL)], ev.at[pl.ds(0, _TAIL)])
        pltpu.sync_copy(b_hbm.at[pl.ds(_TAILB, _TAIL)], bt)
        weight_rows(_TAIL)
        pltpu.sync_copy(xv.at[pl.ds(0, _TAIL), :], accs.at[bt], add=True)
        pltpu.sync_copy(dv.at[pl.ds(0, _TAIL), :], dens.at[bt], add=True)

    plsc.subcore_barrier()

    @pl.when(sid == 0)
    def _writeout():
        pltpu.sync_copy(accs, acc_hbm.at[cid])
        pltpu.sync_copy(dens, den_hbm.at[cid])


def _tc_finish(acc_ref, den_ref, out_ref):
    a = acc_ref[0] + acc_ref[1]
    d = den_ref[0, :, 0:1] + den_ref[1, :, 0:1]
    out_ref[...] = a / (d + 1e-16)


def kernel(x, W1, b1, W2, batch):
    nb = _N // _B
    b1r = b1.reshape(1, _D)
    w2t = W2.reshape(1, _D)
    bi = batch.astype(jnp.int32)

    e3 = pl.pallas_call(
        _tc_scores,
        grid=(nb,),
        in_specs=[
            pl.BlockSpec((_B, _D), lambda i: (i, 0)),
            pl.BlockSpec((_D, _D), lambda i: (0, 0)),
            pl.BlockSpec((1, _D), lambda i: (0, 0)),
            pl.BlockSpec((1, _D), lambda i: (0, 0)),
        ],
        out_specs=pl.BlockSpec((1, 1, _B), lambda i: (i, 0, 0)),
        out_shape=jax.ShapeDtypeStruct((nb, 1, _B), jnp.float32),
    )(x, W1, b1r, w2t)
    e1 = e3.reshape(_N)

    mesh = plsc.VectorSubcoreMesh(core_axis_name="c", subcore_axis_name="s")
    acc2, den2 = pl.kernel(
        _sc_pool,
        out_type=[
            jax.ShapeDtypeStruct((2, _S, _D), jnp.float32),
            jax.ShapeDtypeStruct((2, _S, _D), jnp.float32),
        ],
        mesh=mesh,
        scratch_types=[
            pltpu.VMEM((_R, _D), jnp.float32),
            pltpu.VMEM((_R + 16,), jnp.float32),
            pltpu.VMEM((_R,), jnp.int32),
            pltpu.VMEM((_R, _D), jnp.float32),
            pltpu.VMEM((_TAIL,), jnp.int32),
            pltpu.VMEM((16, _D), jnp.float32),
            pltpu.VMEM_SHARED((_S, _D), jnp.float32),
            pltpu.VMEM_SHARED((_S, _D), jnp.float32),
        ],
    )(x, e1, bi)

    return pl.pallas_call(
        _tc_finish,
        in_specs=[
            pl.BlockSpec((2, _S, _D), lambda: (0, 0, 0)),
            pl.BlockSpec((2, _S, _D), lambda: (0, 0, 0)),
        ],
        out_specs=pl.BlockSpec((_S, _D), lambda: (0, 0)),
        out_shape=jax.ShapeDtypeStruct((_S, _D), jnp.float32),
    )(acc2, den2)


# speedup vs baseline: 1.2106x; 1.2106x over previous
import jax
import jax.numpy as jnp
from jax import lax
from jax.experimental import pallas as pl
from jax.experimental.pallas import tpu as pltpu
from jax.experimental.pallas import tpu_sc as plsc

_N = 100000
_D = 128
_S = 256
_B = 2000
_R = 128
_NCH = _N // _R
_TAILB = _NCH * _R
_TAIL = _N - _TAILB
_NW = 32


def _tc_scores(x_ref, w1_ref, b1_ref, w2_ref, batch_ref, e_ref, deno_ref, den_ref):
    i = pl.program_id(0)
    nb = pl.num_programs(0)

    @pl.when(i == 0)
    def _init():
        den_ref[...] = jnp.zeros_like(den_ref)

    x = x_ref[...]
    h = jnp.tanh(
        jnp.dot(x, w1_ref[...], preferred_element_type=jnp.float32) + b1_ref[...]
    )
    st = lax.dot_general(
        w2_ref[...], h, (((1,), (1,)), ((), ())), preferred_element_type=jnp.float32
    )
    c = jnp.sum(jnp.abs(w2_ref[...]))
    e = jnp.exp(st - c)
    e_ref[...] = e.reshape(1, 1, _B)

    seg = batch_ref[...].reshape(1, _B)
    rows = lax.broadcasted_iota(jnp.int32, (_S, _B), 0)
    Pt = jnp.where(seg == rows, e, 0.0)
    den_ref[:, 0:1] += jnp.sum(Pt, axis=1, keepdims=True)

    @pl.when(i == nb - 1)
    def _fin():
        deno_ref[...] = den_ref[...]


def _sc_pool(x_hbm, e_hbm, b_hbm, acc_hbm, xv, ev, bv, bt, zv, accs):
    cid = lax.axis_index("c")
    sid = lax.axis_index("s")
    wid = sid * 2 + cid

    z16 = jnp.zeros((16,), jnp.float32)
    for r in range(16):
        for q in range(8):
            zv[r, pl.ds(16 * q, 16)] = z16
    pltpu.sync_copy(zv, accs.at[pl.ds(sid * 16, 16), :])
    plsc.subcore_barrier()

    nch_w = (_NCH - 1 - wid) // _NW + 1

    def weight_rows(nrows):
        @plsc.parallel_loop(0, nrows, step=1, unroll=4)
        def _rows(i):
            ei = ev[pl.ds(i, 16)][0]
            es = jnp.full((16,), ei, jnp.float32)
            for q in range(8):
                sl = pl.ds(16 * q, 16)
                xv[i, sl] = xv[i, sl] * es

    def chunk_body(k, carry):
        base = (wid + k * _NW) * _R
        pltpu.sync_copy(x_hbm.at[pl.ds(base, _R), :], xv)
        pltpu.sync_copy(e_hbm.at[pl.ds(base, _R)], ev.at[pl.ds(0, _R)])
        pltpu.sync_copy(b_hbm.at[pl.ds(base, _R)], bv)
        weight_rows(_R)
        pltpu.sync_copy(xv, accs.at[bv], add=True)
        return carry

    lax.fori_loop(0, nch_w, chunk_body, 0)

    @pl.when(wid == 0)
    def _tail():
        pltpu.sync_copy(x_hbm.at[pl.ds(_TAILB, _TAIL), :], xv.at[pl.ds(0, _TAIL), :])
        pltpu.sync_copy(e_hbm.at[pl.ds(_TAILB, _TAIL)], ev.at[pl.ds(0, _TAIL)])
        pltpu.sync_copy(b_hbm.at[pl.ds(_TAILB, _TAIL)], bt)
        weight_rows(_TAIL)
        pltpu.sync_copy(xv.at[pl.ds(0, _TAIL), :], accs.at[bt], add=True)

    plsc.subcore_barrier()

    @pl.when(sid == 0)
    def _writeout():
        pltpu.sync_copy(accs, acc_hbm.at[cid])


def _tc_finish(acc_ref, den_ref, out_ref):
    a = acc_ref[0] + acc_ref[1]
    out_ref[...] = a / (den_ref[:, 0:1] + 1e-16)


def kernel(x, W1, b1, W2, batch):
    nb = _N // _B
    b1r = b1.reshape(1, _D)
    w2t = W2.reshape(1, _D)
    bi = batch.astype(jnp.int32)
    batch3 = bi.reshape(nb, 1, _B)

    e3, den = pl.pallas_call(
        _tc_scores,
        grid=(nb,),
        in_specs=[
            pl.BlockSpec((_B, _D), lambda i: (i, 0)),
            pl.BlockSpec((_D, _D), lambda i: (0, 0)),
            pl.BlockSpec((1, _D), lambda i: (0, 0)),
            pl.BlockSpec((1, _D), lambda i: (0, 0)),
            pl.BlockSpec((1, 1, _B), lambda i: (i, 0, 0)),
        ],
        out_specs=[
            pl.BlockSpec((1, 1, _B), lambda i: (i, 0, 0)),
            pl.BlockSpec((_S, 8), lambda i: (0, 0)),
        ],
        out_shape=[
            jax.ShapeDtypeStruct((nb, 1, _B), jnp.float32),
            jax.ShapeDtypeStruct((_S, 8), jnp.float32),
        ],
        scratch_shapes=[pltpu.VMEM((_S, 8), jnp.float32)],
        compiler_params=pltpu.CompilerParams(
            dimension_semantics=("arbitrary",),
        ),
    )(x, W1, b1r, w2t, batch3)
    e1 = e3.reshape(_N)

    mesh = plsc.VectorSubcoreMesh(core_axis_name="c", subcore_axis_name="s")
    acc2 = pl.kernel(
        _sc_pool,
        out_type=jax.ShapeDtypeStruct((2, _S, _D), jnp.float32),
        mesh=mesh,
        scratch_types=[
            pltpu.VMEM((_R, _D), jnp.float32),
            pltpu.VMEM((_R + 16,), jnp.float32),
            pltpu.VMEM((_R,), jnp.int32),
            pltpu.VMEM((_TAIL,), jnp.int32),
            pltpu.VMEM((16, _D), jnp.float32),
            pltpu.VMEM_SHARED((_S, _D), jnp.float32),
        ],
    )(x, e1, bi)

    return pl.pallas_call(
        _tc_finish,
        in_specs=[
            pl.BlockSpec((2, _S, _D), lambda: (0, 0, 0)),
            pl.BlockSpec((_S, 8), lambda: (0, 0)),
        ],
        out_specs=pl.BlockSpec((_S, _D), lambda: (0, 0)),
        out_shape=jax.ShapeDtypeStruct((_S, _D), jnp.float32),
    )(acc2, den)


# speedup vs baseline: 3.3282x; 2.7491x over previous
import jax
import jax.numpy as jnp
from jax import lax
from jax.experimental import pallas as pl
from jax.experimental.pallas import tpu as pltpu

_N = 100000
_D = 128
_S = 256
_B = 2000


def _tc_kernel(x_ref, w1_ref, b1_ref, w2_ref, batch_ref, out_ref, acc_ref, den_ref):
    i = pl.program_id(0)
    nb = pl.num_programs(0)

    @pl.when(i == 0)
    def _init():
        acc_ref[...] = jnp.zeros_like(acc_ref)
        den_ref[...] = jnp.zeros_like(den_ref)

    x = x_ref[...]
    w2t = w2_ref[...]
    h = jnp.tanh(
        jnp.dot(x, w1_ref[...], preferred_element_type=jnp.float32) + b1_ref[...]
    )
    st = lax.dot_general(
        w2t, h, (((1,), (1,)), ((), ())), preferred_element_type=jnp.float32
    )
    c = jnp.sum(jnp.abs(w2t))
    e = jnp.exp(st - c)

    seg = batch_ref[...].reshape(1, _B)
    rows = lax.broadcasted_iota(jnp.int32, (_S, _B), 0)
    Pt = jnp.where(seg == rows, e, 0.0)

    acc_ref[...] += lax.dot_general(
        Pt, x, (((1,), (0,)), ((), ())), preferred_element_type=jnp.float32
    )
    den_ref[:, 0:1] += jnp.sum(Pt, axis=1, keepdims=True)

    @pl.when(i == nb - 1)
    def _fin():
        out_ref[...] = acc_ref[...] / (den_ref[:, 0:1] + 1e-16)


def kernel(x, W1, b1, W2, batch):
    nb = _N // _B
    batch2 = batch.astype(jnp.int32).reshape(nb, 1, _B)
    b1r = b1.reshape(1, _D)
    w2t = W2.reshape(1, _D)
    return pl.pallas_call(
        _tc_kernel,
        grid=(nb,),
        in_specs=[
            pl.BlockSpec((_B, _D), lambda i: (i, 0)),
            pl.BlockSpec((_D, _D), lambda i: (0, 0)),
            pl.BlockSpec((1, _D), lambda i: (0, 0)),
            pl.BlockSpec((1, _D), lambda i: (0, 0)),
            pl.BlockSpec((1, 1, _B), lambda i: (i, 0, 0)),
        ],
        out_specs=pl.BlockSpec((_S, _D), lambda i: (0, 0)),
        out_shape=jax.ShapeDtypeStruct((_S, _D), jnp.float32),
        scratch_shapes=[
            pltpu.VMEM((_S, _D), jnp.float32),
            pltpu.VMEM((_S, 8), jnp.float32),
        ],
        compiler_params=pltpu.CompilerParams(
            dimension_semantics=("arbitrary",),
        ),
    )(x, W1, b1r, w2t, batch2)
